# baseline (speedup 1.0000x reference)
import jax
import jax.numpy as jnp
from jax import lax
from jax.experimental import pallas as pl
from jax.experimental.pallas import tpu as pltpu
from jax.experimental.pallas import tpu_sc as plsc

N = 10000
D = 128
DH = D // 2
E = 320000
NPAD = 10240
NC = 2
NS = 16
NW = NC * NS
EB = 125
EPW = E // NW
BPW = EPW // EB
KB = 2
NG = BPW // KB
GP = NG // 2
RPW = NPAD // NS
ZR = 32
RB = 2000
NB = N // RB


def _agg_phase(table, src_buf, dst_buf, rows, ones_v, acc, cnt,
               gsems, ssems, with_cnt):
    def _gather(g, b, s):
        j = g * KB + b
        pltpu.async_copy(table.at[src_buf.at[j]], rows[s][b], gsems[s][b])

    def _scatter(g, b, s):
        j = g * KB + b
        pltpu.make_async_copy(table.at[src_buf.at[j]], rows[s][b],
                              gsems[s][b]).wait()
        pltpu.async_copy(rows[s][b], acc.at[dst_buf.at[j]], ssems[s],
                         add=True)
        if with_cnt:
            pltpu.async_copy(ones_v, cnt.at[dst_buf.at[j]], ssems[s],
                             add=True)

    def _drain(g, b, s):
        j = g * KB + b
        pltpu.make_async_copy(rows[s][b], acc.at[dst_buf.at[j]],
                              ssems[s]).wait()
        if with_cnt:
            pltpu.make_async_copy(ones_v, cnt.at[dst_buf.at[j]],
                                  ssems[s]).wait()

    for b in range(KB):
        _gather(0, b, 0)
    for b in range(KB):
        _gather(1, b, 1)

    def body(gp, carry):
        ge = 2 * gp
        go = ge + 1
        for b in range(KB):
            _scatter(ge, b, 0)
        for b in range(KB):
            _scatter(go, b, 1)
        for b in range(KB):
            _drain(ge, b, 0)
        for b in range(KB):
            _gather(ge + 2, b, 0)
        for b in range(KB):
            _drain(go, b, 1)
        for b in range(KB):
            _gather(go + 2, b, 1)
        return carry

    lax.fori_loop(0, GP - 1, body, 0)
    ge = 2 * (GP - 1)
    for b in range(KB):
        _scatter(ge, b, 0)
    for b in range(KB):
        _scatter(ge + 1, b, 1)
    for b in range(KB):
        _drain(ge, b, 0)
    for b in range(KB):
        _drain(ge + 1, b, 1)


def _sc_agg_body(tabA, tabB, ei, aggpA, aggpB, cntp,
                 src_buf, dst_buf,
                 r00, r01, r10, r11,
                 zbuf, zcnt, ones_v, acc, cnt,
                 g00, g01, g10, g11, s0, s1):
    rows = ((r00, r01), (r10, r11))
    gsems = ((g00, g01), (g10, g11))
    ssems = (s0, s1)
    cid = lax.axis_index("c")
    sid = lax.axis_index("s")
    wid = sid * NC + cid
    r0 = sid * RPW
    z16 = jnp.zeros((16,), jnp.float32)
    o16 = jnp.ones((16,), jnp.float32)

    def _zfill(i, c):
        for k in range(DH // 16):
            zbuf[i, pl.ds(16 * k, 16)] = z16
        return c

    lax.fori_loop(0, ZR, _zfill, 0)

    def _zcfill(i, c):
        zcnt[pl.ds(16 * i, 16)] = z16
        return c

    lax.fori_loop(0, RPW // 16, _zcfill, 0)

    def _ofill(i, c):
        ones_v[pl.ds(16 * i, 16)] = o16
        return c

    lax.fori_loop(0, 8, _ofill, 0)

    def _zero_acc():
        for k in range(RPW // ZR):
            pltpu.sync_copy(zbuf, acc.at[pl.ds(r0 + ZR * k, ZR)])

    _zero_acc()
    pltpu.sync_copy(zcnt, cnt.at[pl.ds(r0, RPW)])
    pltpu.sync_copy(ei.at[0, wid], src_buf)
    pltpu.sync_copy(ei.at[1, wid], dst_buf)
    plsc.subcore_barrier()

    _agg_phase(tabA, src_buf, dst_buf, rows, ones_v.at[pl.ds(0, EB)],
               acc, cnt, gsems, ssems, with_cnt=True)
    plsc.subcore_barrier()
    pltpu.sync_copy(acc.at[pl.ds(r0, RPW)], aggpA.at[cid, pl.ds(r0, RPW)])
    pltpu.sync_copy(cnt.at[pl.ds(r0, RPW)], cntp.at[cid, 0, pl.ds(r0, RPW)])
    _zero_acc()
    plsc.subcore_barrier()

    _agg_phase(tabB, src_buf, dst_buf, rows, ones_v.at[pl.ds(0, EB)],
               acc, cnt, gsems, ssems, with_cnt=False)
    plsc.subcore_barrier()
    pltpu.sync_copy(acc.at[pl.ds(r0, RPW)], aggpB.at[cid, pl.ds(r0, RPW)])


def _sc_aggregate(tabA, tabB, ei):
    return pl.kernel(
        _sc_agg_body,
        out_type=(jax.ShapeDtypeStruct((NC, NPAD, DH), jnp.float32),
                  jax.ShapeDtypeStruct((NC, NPAD, DH), jnp.float32),
                  jax.ShapeDtypeStruct((NC, 1, NPAD), jnp.float32)),
        mesh=plsc.VectorSubcoreMesh(core_axis_name="c", subcore_axis_name="s"),
        compiler_params=pltpu.CompilerParams(use_tc_tiling_on_sc=False),
        scratch_types=[
            pltpu.VMEM((BPW, EB), jnp.int32),
            pltpu.VMEM((BPW, EB), jnp.int32),
        ] + [pltpu.VMEM((EB, DH), jnp.float32) for _ in range(2 * KB)]
        + [
            pltpu.VMEM((ZR, DH), jnp.float32),
            pltpu.VMEM((RPW,), jnp.float32),
            pltpu.VMEM((128,), jnp.float32),
            pltpu.VMEM_SHARED((NPAD, DH), jnp.float32),
            pltpu.VMEM_SHARED((NPAD,), jnp.float32),
        ] + [pltpu.SemaphoreType.DMA for _ in range(2 * KB + 2)],
    )(tabA, tabB, ei)


def _mm_t(a, w):
    return lax.dot_general(a, w, (((1,), (1,)), ((), ())))


def _sage_lin(aggpA, aggpB, cntp, wl, bl, hA, hB, wr):
    cnt = jnp.maximum(cntp[0] + cntp[1], 1.0)
    meanA = (aggpA[0] + aggpA[1]) / cnt
    meanB = (aggpB[0] + aggpB[1]) / cnt
    return (_mm_t(meanA, wl[:, :DH]) + _mm_t(meanB, wl[:, DH:])
            + bl[...][None, :]
            + _mm_t(hA[...], wr[:, :DH]) + _mm_t(hB[...], wr[:, DH:]))


def _lin_body(aggpA, aggpB, cntp, xA, xB, wl, bl, wr, h_out, s1_out, s2_out):
    h = _sage_lin(aggpA, aggpB, cntp, wl, bl, xA, xB, wr)
    h_out[...] = h

    @pl.when(pl.program_id(0) == 0)
    def _init():
        s1_out[...] = jnp.zeros_like(s1_out)
        s2_out[...] = jnp.zeros_like(s2_out)

    s1_out[...] += jnp.sum(h, axis=0, keepdims=True)
    s2_out[...] += jnp.sum(h * h, axis=0, keepdims=True)


def _bn_relu_body(h, s1, s2, gamma, beta, h2A, h2B):
    mu = s1[...] / N
    var = s2[...] / N - mu * mu
    inv = gamma[...][None, :] / jnp.sqrt(var + 1e-5)
    h2 = jnp.maximum((h[...] - mu) * inv + beta[...][None, :], 0.0)
    h2A[...] = h2[:, :DH]
    h2B[...] = h2[:, DH:]


def _lin2_body(aggpA, aggpB, cntp, hA, hB, wl, bl, wr, out):
    out[...] = _sage_lin(aggpA, aggpB, cntp, wl, bl, hA, hB, wr)


_ROW = pl.BlockSpec((RB, D), lambda i: (i, 0))
_ROWH = pl.BlockSpec((RB, DH), lambda i: (i, 0))
_AGGP = pl.BlockSpec((NC, RB, DH), lambda i: (0, i, 0))
_CNTP = pl.BlockSpec((NC, RB, 1), lambda i: (0, i, 0))
_WMAT = pl.BlockSpec((D, D), lambda i: (0, 0))
_WVEC = pl.BlockSpec((D,), lambda i: (0,))
_STAT = pl.BlockSpec((1, D), lambda i: (0, 0))


def kernel(x, edge_index, Wl1, bl1, Wr1, gamma1, beta1, Wl2, bl2, Wr2):
    ei = edge_index.reshape(2, NW, BPW, EB)
    xA = x[:, :DH]
    xB = x[:, DH:]

    aggpA1, aggpB1, cntp = _sc_aggregate(xA, xB, ei)
    cntp = cntp.reshape(NC, NPAD, 1)

    h, s1, s2 = pl.pallas_call(
        _lin_body,
        grid=(NB,),
        in_specs=[_AGGP, _AGGP, _CNTP, _ROWH, _ROWH, _WMAT, _WVEC, _WMAT],
        out_specs=[_ROW, _STAT, _STAT],
        out_shape=[jax.ShapeDtypeStruct((N, D), jnp.float32),
                   jax.ShapeDtypeStruct((1, D), jnp.float32),
                   jax.ShapeDtypeStruct((1, D), jnp.float32)],
    )(aggpA1, aggpB1, cntp, xA, xB, Wl1, bl1, Wr1)

    h2A, h2B = pl.pallas_call(
        _bn_relu_body,
        grid=(NB,),
        in_specs=[_ROW, _STAT, _STAT, _WVEC, _WVEC],
        out_specs=[_ROWH, _ROWH],
        out_shape=[jax.ShapeDtypeStruct((N, DH), jnp.float32),
                   jax.ShapeDtypeStruct((N, DH), jnp.float32)],
    )(h, s1, s2, gamma1, beta1)

    aggpA2, aggpB2, _ = _sc_aggregate(h2A, h2B, ei)

    out = pl.pallas_call(
        _lin2_body,
        grid=(NB,),
        in_specs=[_AGGP, _AGGP, _CNTP, _ROWH, _ROWH, _WMAT, _WVEC, _WMAT],
        out_specs=_ROW,
        out_shape=jax.ShapeDtypeStruct((N, D), jnp.float32),
    )(aggpA2, aggpB2, cntp, h2A, h2B, Wl2, bl2, Wr2)
    return out


# speedup vs baseline: 1.0677x; 1.0677x over previous
import functools

import jax
import jax.numpy as jnp
from jax import lax
from jax.experimental import pallas as pl
from jax.experimental.pallas import tpu as pltpu
from jax.experimental.pallas import tpu_sc as plsc

N = 10000
D = 128
DH = D // 2
E = 320000
NPAD = 10240
NC = 2
NS = 16
NW = NC * NS
EB = 125
EPW = E // NW
BPW = EPW // EB
KPW = BPW // 2
RPW = NPAD // NS
ZR = 32
RB = 2000
NB = N // RB


def _agg_phase(table, src_buf, dst_buf, rows0, rows1, ones_v, acc, cnt,
               g0, g1, with_cnt):
    pltpu.async_copy(table.at[src_buf.at[0]], rows0, g0)

    def body(k, carry):
        j0 = 2 * k
        j1 = j0 + 1
        pltpu.async_copy(table.at[src_buf.at[j1]], rows1, g1)
        pltpu.make_async_copy(table.at[src_buf.at[j0]], rows0, g0).wait()
        pltpu.sync_copy(rows0, acc.at[dst_buf.at[j0]], add=True)
        if with_cnt:
            pltpu.sync_copy(ones_v, cnt.at[dst_buf.at[j0]], add=True)
        jn = jnp.minimum(j0 + 2, BPW - 2)
        pltpu.async_copy(table.at[src_buf.at[jn]], rows0, g0)
        pltpu.make_async_copy(table.at[src_buf.at[j1]], rows1, g1).wait()
        pltpu.sync_copy(rows1, acc.at[dst_buf.at[j1]], add=True)
        if with_cnt:
            pltpu.sync_copy(ones_v, cnt.at[dst_buf.at[j1]], add=True)
        return carry

    lax.fori_loop(0, KPW, body, 0)
    pltpu.make_async_copy(table.at[src_buf.at[0]], rows0, g0).wait()


def _sc_agg_body(tabA, tabB, ei, aggpA, aggpB, cntp,
                 src_buf, dst_buf, rows0, rows1, zbuf, zcnt, ones_v,
                 acc, cnt, g0, g1, *, with_cnt):
    cid = lax.axis_index("c")
    sid = lax.axis_index("s")
    wid = sid * NC + cid
    r0 = sid * RPW
    z16 = jnp.zeros((16,), jnp.float32)
    o16 = jnp.ones((16,), jnp.float32)

    def _zfill(i, c):
        for k in range(DH // 16):
            zbuf[i, pl.ds(16 * k, 16)] = z16
        return c

    lax.fori_loop(0, ZR, _zfill, 0)

    if with_cnt:
        def _zcfill(i, c):
            zcnt[pl.ds(16 * i, 16)] = z16
            return c

        lax.fori_loop(0, RPW // 16, _zcfill, 0)

        def _ofill(i, c):
            ones_v[pl.ds(16 * i, 16)] = o16
            return c

        lax.fori_loop(0, 8, _ofill, 0)

    def _zero_acc():
        for k in range(RPW // ZR):
            pltpu.sync_copy(zbuf, acc.at[pl.ds(r0 + ZR * k, ZR)])

    _zero_acc()
    if with_cnt:
        pltpu.sync_copy(zcnt, cnt.at[pl.ds(r0, RPW)])
    pltpu.sync_copy(ei.at[0, wid], src_buf)
    pltpu.sync_copy(ei.at[1, wid], dst_buf)
    plsc.subcore_barrier()

    _agg_phase(tabA, src_buf, dst_buf, rows0, rows1,
               ones_v.at[pl.ds(0, EB)], acc, cnt, g0, g1, with_cnt=with_cnt)
    plsc.subcore_barrier()
    pltpu.sync_copy(acc.at[pl.ds(r0, RPW)], aggpA.at[cid, pl.ds(r0, RPW)])
    if with_cnt:
        pltpu.sync_copy(cnt.at[pl.ds(r0, RPW)], cntp.at[cid, 0, pl.ds(r0, RPW)])
    _zero_acc()
    plsc.subcore_barrier()

    _agg_phase(tabB, src_buf, dst_buf, rows0, rows1,
               ones_v.at[pl.ds(0, EB)], acc, cnt, g0, g1, with_cnt=False)
    plsc.subcore_barrier()
    pltpu.sync_copy(acc.at[pl.ds(r0, RPW)], aggpB.at[cid, pl.ds(r0, RPW)])


def _sc_aggregate(tabA, tabB, ei, with_cnt):
    out_type = [jax.ShapeDtypeStruct((NC, NPAD, DH), jnp.float32),
                jax.ShapeDtypeStruct((NC, NPAD, DH), jnp.float32)]
    if with_cnt:
        out_type.append(jax.ShapeDtypeStruct((NC, 1, NPAD), jnp.float32))
        body = functools.partial(_sc_agg_body, with_cnt=True)
    else:
        def body(tA, tB, e, aggpA, aggpB, *scratch):
            return _sc_agg_body(tA, tB, e, aggpA, aggpB, None, *scratch,
                                with_cnt=False)
    res = pl.kernel(
        body,
        out_type=tuple(out_type),
        mesh=plsc.VectorSubcoreMesh(core_axis_name="c", subcore_axis_name="s"),
        compiler_params=pltpu.CompilerParams(use_tc_tiling_on_sc=False),
        scratch_types=[
            pltpu.VMEM((BPW, EB), jnp.int32),
            pltpu.VMEM((BPW, EB), jnp.int32),
            pltpu.VMEM((EB, DH), jnp.float32),
            pltpu.VMEM((EB, DH), jnp.float32),
            pltpu.VMEM((ZR, DH), jnp.float32),
            pltpu.VMEM((RPW,), jnp.float32),
            pltpu.VMEM((128,), jnp.float32),
            pltpu.VMEM_SHARED((NPAD, DH), jnp.float32),
            pltpu.VMEM_SHARED((NPAD,), jnp.float32),
            pltpu.SemaphoreType.DMA,
            pltpu.SemaphoreType.DMA,
        ],
    )(tabA, tabB, ei)
    return res if with_cnt else (res[0], res[1], None)


def _mm_t(a, w):
    return lax.dot_general(a, w, (((1,), (1,)), ((), ())))


def _sage_lin(aggpA, aggpB, cntp, wl, bl, hA, hB, wr):
    cnt = jnp.maximum(cntp[0] + cntp[1], 1.0)
    meanA = (aggpA[0] + aggpA[1]) / cnt
    meanB = (aggpB[0] + aggpB[1]) / cnt
    return (_mm_t(meanA, wl[:, :DH]) + _mm_t(meanB, wl[:, DH:])
            + bl[...][None, :]
            + _mm_t(hA[...], wr[:, :DH]) + _mm_t(hB[...], wr[:, DH:]))


def _lin_body(aggpA, aggpB, cntp, xA, xB, wl, bl, wr, h_out, s1_out, s2_out):
    h = _sage_lin(aggpA, aggpB, cntp, wl, bl, xA, xB, wr)
    h_out[...] = h

    @pl.when(pl.program_id(0) == 0)
    def _init():
        s1_out[...] = jnp.zeros_like(s1_out)
        s2_out[...] = jnp.zeros_like(s2_out)

    s1_out[...] += jnp.sum(h, axis=0, keepdims=True)
    s2_out[...] += jnp.sum(h * h, axis=0, keepdims=True)


def _bn_relu_body(h, s1, s2, gamma, beta, h2A, h2B):
    mu = s1[...] / N
    var = s2[...] / N - mu * mu
    inv = gamma[...][None, :] / jnp.sqrt(var + 1e-5)
    h2 = jnp.maximum((h[...] - mu) * inv + beta[...][None, :], 0.0)
    h2A[...] = h2[:, :DH]
    h2B[...] = h2[:, DH:]


def _lin2_body(aggpA, aggpB, cntp, hA, hB, wl, bl, wr, out):
    out[...] = _sage_lin(aggpA, aggpB, cntp, wl, bl, hA, hB, wr)


_ROW = pl.BlockSpec((RB, D), lambda i: (i, 0))
_ROWH = pl.BlockSpec((RB, DH), lambda i: (i, 0))
_AGGP = pl.BlockSpec((NC, RB, DH), lambda i: (0, i, 0))
_CNTP = pl.BlockSpec((NC, RB, 1), lambda i: (0, i, 0))
_WMAT = pl.BlockSpec((D, D), lambda i: (0, 0))
_WVEC = pl.BlockSpec((D,), lambda i: (0,))
_STAT = pl.BlockSpec((1, D), lambda i: (0, 0))


def kernel(x, edge_index, Wl1, bl1, Wr1, gamma1, beta1, Wl2, bl2, Wr2):
    ei = edge_index.reshape(2, NW, BPW, EB)
    xA = x[:, :DH]
    xB = x[:, DH:]

    aggpA1, aggpB1, cntp = _sc_aggregate(xA, xB, ei, with_cnt=True)
    cntp = cntp.reshape(NC, NPAD, 1)

    h, s1, s2 = pl.pallas_call(
        _lin_body,
        grid=(NB,),
        in_specs=[_AGGP, _AGGP, _CNTP, _ROWH, _ROWH, _WMAT, _WVEC, _WMAT],
        out_specs=[_ROW, _STAT, _STAT],
        out_shape=[jax.ShapeDtypeStruct((N, D), jnp.float32),
                   jax.ShapeDtypeStruct((1, D), jnp.float32),
                   jax.ShapeDtypeStruct((1, D), jnp.float32)],
    )(aggpA1, aggpB1, cntp, xA, xB, Wl1, bl1, Wr1)

    h2A, h2B = pl.pallas_call(
        _bn_relu_body,
        grid=(NB,),
        in_specs=[_ROW, _STAT, _STAT, _WVEC, _WVEC],
        out_specs=[_ROWH, _ROWH],
        out_shape=[jax.ShapeDtypeStruct((N, DH), jnp.float32),
                   jax.ShapeDtypeStruct((N, DH), jnp.float32)],
    )(h, s1, s2, gamma1, beta1)

    aggpA2, aggpB2, _ = _sc_aggregate(h2A, h2B, ei, with_cnt=False)

    out = pl.pallas_call(
        _lin2_body,
        grid=(NB,),
        in_specs=[_AGGP, _AGGP, _CNTP, _ROWH, _ROWH, _WMAT, _WVEC, _WMAT],
        out_specs=_ROW,
        out_shape=jax.ShapeDtypeStruct((N, D), jnp.float32),
    )(aggpA2, aggpB2, cntp, h2A, h2B, Wl2, bl2, Wr2)
    return out


# speedup vs baseline: 1.0695x; 1.0017x over previous
import functools

import jax
import jax.numpy as jnp
from jax import lax
from jax.experimental import pallas as pl
from jax.experimental.pallas import tpu as pltpu
from jax.experimental.pallas import tpu_sc as plsc

N = 10000
D = 128
DH = D // 2
E = 320000
NPAD = 10240
NC = 2
NS = 16
NW = NC * NS
EB = 125
EPW = E // NW
BPW = EPW // EB
KPW = BPW // 2
RPW = NPAD // NS
ZR = 32
RB = 2000
NB = N // RB


def _agg_phase(table, src_buf, dst_buf, rows0, rows1, ones_v, acc, cnt,
               g0, g1, with_cnt):
    pltpu.async_copy(table.at[src_buf.at[0]], rows0, g0)

    def body(k, carry):
        j0 = 2 * k
        j1 = j0 + 1
        pltpu.async_copy(table.at[src_buf.at[j1]], rows1, g1)
        pltpu.make_async_copy(table.at[src_buf.at[j0]], rows0, g0).wait()
        pltpu.sync_copy(rows0, acc.at[dst_buf.at[j0]], add=True)
        if with_cnt:
            pltpu.sync_copy(ones_v, cnt.at[dst_buf.at[j0]], add=True)
        jn = jnp.minimum(j0 + 2, BPW - 2)
        pltpu.async_copy(table.at[src_buf.at[jn]], rows0, g0)
        pltpu.make_async_copy(table.at[src_buf.at[j1]], rows1, g1).wait()
        pltpu.sync_copy(rows1, acc.at[dst_buf.at[j1]], add=True)
        if with_cnt:
            pltpu.sync_copy(ones_v, cnt.at[dst_buf.at[j1]], add=True)
        return carry

    lax.fori_loop(0, KPW, body, 0)
    pltpu.make_async_copy(table.at[src_buf.at[0]], rows0, g0).wait()


def _sc_agg_body(tabA, tabB, ei, aggpA, aggpB, cntp,
                 src_buf, dst_buf, rows0, rows1, zbuf, zcnt, ones_v,
                 acc, cnt, g0, g1, *, with_cnt):
    cid = lax.axis_index("c")
    sid = lax.axis_index("s")
    wid = sid * NC + cid
    r0 = sid * RPW
    z16 = jnp.zeros((16,), jnp.float32)
    o16 = jnp.ones((16,), jnp.float32)

    def _zfill(i, c):
        for k in range(DH // 16):
            zbuf[i, pl.ds(16 * k, 16)] = z16
        return c

    lax.fori_loop(0, ZR, _zfill, 0)

    if with_cnt:
        def _zcfill(i, c):
            zcnt[pl.ds(16 * i, 16)] = z16
            return c

        lax.fori_loop(0, RPW // 16, _zcfill, 0)

        def _ofill(i, c):
            ones_v[pl.ds(16 * i, 16)] = o16
            return c

        lax.fori_loop(0, 8, _ofill, 0)

    def _zero_acc():
        for k in range(RPW // ZR):
            pltpu.sync_copy(zbuf, acc.at[pl.ds(r0 + ZR * k, ZR)])

    _zero_acc()
    if with_cnt:
        pltpu.sync_copy(zcnt, cnt.at[pl.ds(r0, RPW)])
    pltpu.sync_copy(ei.at[0, wid], src_buf)
    pltpu.sync_copy(ei.at[1, wid], dst_buf)
    plsc.subcore_barrier()

    _agg_phase(tabA, src_buf, dst_buf, rows0, rows1,
               ones_v.at[pl.ds(0, EB)], acc, cnt, g0, g1, with_cnt=with_cnt)
    plsc.subcore_barrier()
    pltpu.sync_copy(acc.at[pl.ds(r0, RPW)], aggpA.at[cid, pl.ds(r0, RPW)])
    if with_cnt:
        pltpu.sync_copy(cnt.at[pl.ds(r0, RPW)], cntp.at[cid, 0, pl.ds(r0, RPW)])
    _zero_acc()
    plsc.subcore_barrier()

    _agg_phase(tabB, src_buf, dst_buf, rows0, rows1,
               ones_v.at[pl.ds(0, EB)], acc, cnt, g0, g1, with_cnt=False)
    plsc.subcore_barrier()
    pltpu.sync_copy(acc.at[pl.ds(r0, RPW)], aggpB.at[cid, pl.ds(r0, RPW)])


def _sc_aggregate(tabA, tabB, ei, with_cnt):
    out_type = [jax.ShapeDtypeStruct((NC, NPAD, DH), jnp.float32),
                jax.ShapeDtypeStruct((NC, NPAD, DH), jnp.float32)]
    if with_cnt:
        out_type.append(jax.ShapeDtypeStruct((NC, 1, NPAD), jnp.float32))
        body = functools.partial(_sc_agg_body, with_cnt=True)
    else:
        def body(tA, tB, e, aggpA, aggpB, *scratch):
            return _sc_agg_body(tA, tB, e, aggpA, aggpB, None, *scratch,
                                with_cnt=False)
    res = pl.kernel(
        body,
        out_type=tuple(out_type),
        mesh=plsc.VectorSubcoreMesh(core_axis_name="c", subcore_axis_name="s"),
        compiler_params=pltpu.CompilerParams(use_tc_tiling_on_sc=False),
        scratch_types=[
            pltpu.VMEM((BPW, EB), jnp.int32),
            pltpu.VMEM((BPW, EB), jnp.int32),
            pltpu.VMEM((EB, DH), jnp.float32),
            pltpu.VMEM((EB, DH), jnp.float32),
            pltpu.VMEM((ZR, DH), jnp.float32),
            pltpu.VMEM((RPW,), jnp.float32),
            pltpu.VMEM((128,), jnp.float32),
            pltpu.VMEM_SHARED((NPAD, DH), jnp.float32),
            pltpu.VMEM_SHARED((NPAD,), jnp.float32),
            pltpu.SemaphoreType.DMA,
            pltpu.SemaphoreType.DMA,
        ],
    )(tabA, tabB, ei)
    return res if with_cnt else (res[0], res[1], None)


def _mm_t(a, w):
    return lax.dot_general(a, w, (((1,), (1,)), ((), ())))


def _root_body(hA, hB, wr, bl, out):
    out[...] = (_mm_t(hA[...], wr[:, :DH]) + _mm_t(hB[...], wr[:, DH:])
                + bl[...][None, :])


def _sage_lin(aggpA, aggpB, cntp, wl, hr):
    cnt = jnp.maximum(cntp[0] + cntp[1], 1.0)
    meanA = (aggpA[0] + aggpA[1]) / cnt
    meanB = (aggpB[0] + aggpB[1]) / cnt
    return _mm_t(meanA, wl[:, :DH]) + _mm_t(meanB, wl[:, DH:]) + hr[...]


def _lin_body(aggpA, aggpB, cntp, xr, wl, h_out, s1_out, s2_out):
    h = _sage_lin(aggpA, aggpB, cntp, wl, xr)
    h_out[...] = h

    @pl.when(pl.program_id(0) == 0)
    def _init():
        s1_out[...] = jnp.zeros_like(s1_out)
        s2_out[...] = jnp.zeros_like(s2_out)

    s1_out[...] += jnp.sum(h, axis=0, keepdims=True)
    s2_out[...] += jnp.sum(h * h, axis=0, keepdims=True)


def _bn_relu_body(h, s1, s2, gamma, beta, h2A, h2B):
    mu = s1[...] / N
    var = s2[...] / N - mu * mu
    inv = gamma[...][None, :] / jnp.sqrt(var + 1e-5)
    h2 = jnp.maximum((h[...] - mu) * inv + beta[...][None, :], 0.0)
    h2A[...] = h2[:, :DH]
    h2B[...] = h2[:, DH:]


def _lin2_body(aggpA, aggpB, cntp, hr, wl, out):
    out[...] = _sage_lin(aggpA, aggpB, cntp, wl, hr)


_ROW = pl.BlockSpec((RB, D), lambda i: (i, 0))
_ROWH = pl.BlockSpec((RB, DH), lambda i: (i, 0))
_AGGP = pl.BlockSpec((NC, RB, DH), lambda i: (0, i, 0))
_CNTP = pl.BlockSpec((NC, RB, 1), lambda i: (0, i, 0))
_WMAT = pl.BlockSpec((D, D), lambda i: (0, 0))
_WVEC = pl.BlockSpec((D,), lambda i: (0,))
_STAT = pl.BlockSpec((1, D), lambda i: (0, 0))


def kernel(x, edge_index, Wl1, bl1, Wr1, gamma1, beta1, Wl2, bl2, Wr2):
    ei = edge_index.reshape(2, NW, BPW, EB)
    xA = x[:, :DH]
    xB = x[:, DH:]

    aggpA1, aggpB1, cntp = _sc_aggregate(xA, xB, ei, with_cnt=True)
    cntp = cntp.reshape(NC, NPAD, 1)

    xr = pl.pallas_call(
        _root_body,
        grid=(NB,),
        in_specs=[_ROWH, _ROWH, _WMAT, _WVEC],
        out_specs=_ROW,
        out_shape=jax.ShapeDtypeStruct((N, D), jnp.float32),
    )(xA, xB, Wr1, bl1)

    h, s1, s2 = pl.pallas_call(
        _lin_body,
        grid=(NB,),
        in_specs=[_AGGP, _AGGP, _CNTP, _ROW, _WMAT],
        out_specs=[_ROW, _STAT, _STAT],
        out_shape=[jax.ShapeDtypeStruct((N, D), jnp.float32),
                   jax.ShapeDtypeStruct((1, D), jnp.float32),
                   jax.ShapeDtypeStruct((1, D), jnp.float32)],
    )(aggpA1, aggpB1, cntp, xr, Wl1)

    h2A, h2B = pl.pallas_call(
        _bn_relu_body,
        grid=(NB,),
        in_specs=[_ROW, _STAT, _STAT, _WVEC, _WVEC],
        out_specs=[_ROWH, _ROWH],
        out_shape=[jax.ShapeDtypeStruct((N, DH), jnp.float32),
                   jax.ShapeDtypeStruct((N, DH), jnp.float32)],
    )(h, s1, s2, gamma1, beta1)

    aggpA2, aggpB2, _ = _sc_aggregate(h2A, h2B, ei, with_cnt=False)

    hr2 = pl.pallas_call(
        _root_body,
        grid=(NB,),
        in_specs=[_ROWH, _ROWH, _WMAT, _WVEC],
        out_specs=_ROW,
        out_shape=jax.ShapeDtypeStruct((N, D), jnp.float32),
    )(h2A, h2B, Wr2, bl2)

    out = pl.pallas_call(
        _lin2_body,
        grid=(NB,),
        in_specs=[_AGGP, _AGGP, _CNTP, _ROW, _WMAT],
        out_specs=_ROW,
        out_shape=jax.ShapeDtypeStruct((N, D), jnp.float32),
    )(aggpA2, aggpB2, cntp, hr2, Wl2)
    return out
